# baseline (speedup 1.0000x reference)
import functools

import numpy as np
import jax
import jax.numpy as jnp
from jax import lax
from jax.experimental import pallas as pl
from jax.experimental.pallas import tpu as pltpu
from jax.experimental.pallas import tpu_sc as plsc

WIN = 640
RB = 128
CHUNK = 128
_UPPER = np.triu(np.ones((CHUNK, CHUNK), dtype=np.float32))


def _make_render_body(total: int):
    def body(ns_ref, bkg_ref, u_ref, pk_hbm, rgb_ref, mask_ref, sbuf, sems):
        i = pl.program_id(0)
        n = pl.num_programs(0)
        slot = lax.rem(i, 2)

        def blk_copy(b, sl):
            return pltpu.make_async_copy(
                pk_hbm.at[:, pl.ds(b * RB, RB), :], sbuf.at[sl], sems.at[sl])

        @pl.when(i == 0)
        def _():
            blk_copy(0, 0).start()

        @pl.when(i + 1 < n)
        def _():
            blk_copy(i + 1, lax.rem(i + 1, 2)).start()

        blk_copy(i, slot).wait()

        ns = ns_ref[...]
        steps = ns[:, 0:1]
        off = ns[:, 1:2]
        ws = jnp.minimum(jnp.bitwise_and(off, -8),
                         jnp.bitwise_and(total - WIN, -8))
        ls = off - ws
        sig = sbuf[slot, 0]
        dtv = sbuf[slot, 1]
        iota = lax.broadcasted_iota(jnp.int32, (RB, WIN), 1)
        valid = (iota >= ls) & (iota < ls + steps)
        x = jnp.where(valid, jnp.maximum(sig, 0.0) * dtv, 0.0)
        u = u_ref[...]
        parts = []
        carry = jnp.zeros((RB, 1), jnp.float32)
        for k in range(WIN // CHUNK):
            ck = x[:, k * CHUNK:(k + 1) * CHUNK]
            sk = jnp.dot(ck, u, preferred_element_type=jnp.float32,
                         precision=lax.Precision.HIGHEST) + carry
            carry = sk[:, CHUNK - 1:CHUNK]
            parts.append(sk)
        s = jnp.concatenate(parts, axis=1)
        w = jnp.exp(x - s) - jnp.exp(-s)
        s_end = s[:, WIN - 1:WIN]
        xl = jnp.sum(jnp.where(iota == ls + 511, x, 0.0), axis=1, keepdims=True)
        t_bkg = jnp.exp(xl - s_end)
        mask_ref[...] = 1.0 - jnp.exp(-s_end)
        for c in range(3):
            acc = jnp.sum(w * sbuf[slot, 2 + c], axis=1, keepdims=True)
            rgb_ref[:, c:c + 1] = acc + t_bkg * bkg_ref[0, c]

    return body


def _render(pk, ns, bkg, total):
    n_rays = ns.shape[0]
    grid = (n_rays // RB,)
    return pl.pallas_call(
        _make_render_body(total),
        grid=grid,
        in_specs=[
            pl.BlockSpec((RB, 2), lambda i: (i, 0)),
            pl.BlockSpec((1, 3), lambda i: (0, 0)),
            pl.BlockSpec((CHUNK, CHUNK), lambda i: (0, 0)),
            pl.BlockSpec(memory_space=pltpu.MemorySpace.HBM),
        ],
        out_specs=[
            pl.BlockSpec((RB, 3), lambda i: (i, 0)),
            pl.BlockSpec((RB, 1), lambda i: (i, 0)),
        ],
        out_shape=[
            jax.ShapeDtypeStruct((n_rays, 3), jnp.float32),
            jax.ShapeDtypeStruct((n_rays, 1), jnp.float32),
        ],
        scratch_shapes=[
            pltpu.VMEM((2, 5, RB, WIN), jnp.float32),
            pltpu.SemaphoreType.DMA((2,)),
        ],
    )(ns, bkg, jnp.asarray(_UPPER), pk)


def _sc_pack(sigma, dtv, rad, offs, n_rays, total):
    info = plsc.get_sparse_core_info()
    nw = info.num_cores * info.num_subcores
    rpw = n_rays // nw
    wclamp = (total - WIN) & -8

    @functools.partial(
        pl.kernel,
        out_type=jax.ShapeDtypeStruct((5, n_rays, WIN), jnp.float32),
        mesh=plsc.VectorSubcoreMesh(core_axis_name="c", subcore_axis_name="s"),
        scratch_types=[
            pltpu.VMEM((rpw,), jnp.int32),
            pltpu.VMEM((4, 5, WIN), jnp.float32),
            pltpu.SemaphoreType.DMA,
            pltpu.SemaphoreType.DMA,
        ],
        compiler_params=pltpu.CompilerParams(use_tc_tiling_on_sc=False),
    )
    def pack(sig_hbm, dt_hbm, rad_hbm, offs_hbm, out, offs_v, buf,
             gsem, ssem):
        c = lax.axis_index("c")
        s = lax.axis_index("s")
        wid = s * info.num_cores + c
        base = wid * rpw
        pltpu.sync_copy(offs_hbm.at[pl.ds(base, rpw)], offs_v)

        def outer(g, carry):
            offv = offs_v[pl.ds(g * 16, 16)]
            gathers = {}
            scatters = {}

            def fire(j):
                b = j % 4
                if b in scatters:
                    scatters.pop(b).wait()
                off = offv[j]
                ws = pl.multiple_of(
                    jnp.minimum(jnp.bitwise_and(off, -8), wclamp), 8)
                gathers[b] = [
                    pltpu.async_copy(sig_hbm.at[pl.ds(ws, WIN)],
                                     buf.at[b, 0], gsem),
                    pltpu.async_copy(dt_hbm.at[pl.ds(ws, WIN)],
                                     buf.at[b, 1], gsem),
                    pltpu.async_copy(rad_hbm.at[:, pl.ds(ws, WIN)],
                                     buf.at[b, pl.ds(2, 3)], gsem),
                ]

            fire(0)
            fire(1)
            fire(2)
            for j in range(16):
                if j + 3 < 16:
                    fire(j + 3)
                b = j % 4
                for h in gathers.pop(b):
                    h.wait()
                ray = base + g * 16 + j
                scatters[b] = pltpu.async_copy(buf.at[b], out.at[:, ray, :],
                                               ssem)
            for b in list(scatters):
                scatters.pop(b).wait()
            return carry

        lax.fori_loop(0, rpw // 16, outer, 0)

    return pack(sigma, dtv, rad, offs)


def kernel(sigma, radiance, dt, numsteps_in, bkg_color, inference_only):
    del inference_only
    n_rays = numsteps_in.shape[0]
    total = sigma.shape[0]
    nch = 4
    cr = n_rays // nch
    rad_t = radiance.T
    rgbs, masks = [], []
    for k in range(nch):
        ns_k = lax.slice_in_dim(numsteps_in, k * cr, (k + 1) * cr, axis=0)
        pk = _sc_pack(sigma, dt, rad_t, ns_k[:, 1], cr, total)
        rgb_k, mask_k = _render(pk, ns_k, bkg_color, total)
        rgbs.append(rgb_k)
        masks.append(mask_k)
    rgb = jnp.concatenate(rgbs, axis=0)
    mask = jnp.concatenate(masks, axis=0)
    return rgb, mask.reshape(n_rays)


# speedup vs baseline: 1.0058x; 1.0058x over previous
import functools

import numpy as np
import jax
import jax.numpy as jnp
from jax import lax
from jax.experimental import pallas as pl
from jax.experimental.pallas import tpu as pltpu
from jax.experimental.pallas import tpu_sc as plsc

WIN = 640
RB = 128
CHUNK = 128
_UPPER = np.triu(np.ones((CHUNK, CHUNK), dtype=np.float32))


def _make_render_body(total: int):
    def body(ns_ref, bkg_ref, u_ref, pk_ref, rgb_ref, mask_ref):
        ns = ns_ref[...]
        steps = ns[:, 0:1]
        off = ns[:, 1:2]
        ws = jnp.minimum(jnp.bitwise_and(off, -8),
                         jnp.bitwise_and(total - WIN, -8))
        ls = off - ws
        sig = pk_ref[0]
        dtv = pk_ref[1]
        iota = lax.broadcasted_iota(jnp.int32, (RB, WIN), 1)
        valid = (iota >= ls) & (iota < ls + steps)
        x = jnp.where(valid, jnp.maximum(sig, 0.0) * dtv, 0.0)
        u = u_ref[...]
        parts = []
        carry = jnp.zeros((RB, 1), jnp.float32)
        for k in range(WIN // CHUNK):
            ck = x[:, k * CHUNK:(k + 1) * CHUNK]
            sk = jnp.dot(ck, u, preferred_element_type=jnp.float32,
                         precision=lax.Precision.HIGHEST) + carry
            carry = sk[:, CHUNK - 1:CHUNK]
            parts.append(sk)
        s = jnp.concatenate(parts, axis=1)
        w = jnp.exp(x - s) - jnp.exp(-s)
        s_end = s[:, WIN - 1:WIN]
        xl = jnp.sum(jnp.where(iota == ls + 511, x, 0.0), axis=1, keepdims=True)
        t_bkg = jnp.exp(xl - s_end)
        mask_ref[...] = 1.0 - jnp.exp(-s_end)
        for c in range(3):
            acc = jnp.sum(w * pk_ref[2 + c], axis=1, keepdims=True)
            rgb_ref[:, c:c + 1] = acc + t_bkg * bkg_ref[0, c]

    return body


def _render(pk, ns, bkg, total):
    n_rays = ns.shape[0]
    grid = (n_rays // RB,)
    return pl.pallas_call(
        _make_render_body(total),
        grid=grid,
        in_specs=[
            pl.BlockSpec((RB, 2), lambda i: (i, 0)),
            pl.BlockSpec((1, 3), lambda i: (0, 0)),
            pl.BlockSpec((CHUNK, CHUNK), lambda i: (0, 0)),
            pl.BlockSpec((5, RB, WIN), lambda i: (0, i, 0)),
        ],
        out_specs=[
            pl.BlockSpec((RB, 3), lambda i: (i, 0)),
            pl.BlockSpec((RB, 1), lambda i: (i, 0)),
        ],
        out_shape=[
            jax.ShapeDtypeStruct((n_rays, 3), jnp.float32),
            jax.ShapeDtypeStruct((n_rays, 1), jnp.float32),
        ],
    )(ns, bkg, jnp.asarray(_UPPER), pk)


def _sc_pack(sigma, dtv, rad, offs, n_rays, total):
    info = plsc.get_sparse_core_info()
    nw = info.num_cores * info.num_subcores
    rpw = n_rays // nw
    wclamp = (total - WIN) & -8

    @functools.partial(
        pl.kernel,
        out_type=jax.ShapeDtypeStruct((5, n_rays, WIN), jnp.float32),
        mesh=plsc.VectorSubcoreMesh(core_axis_name="c", subcore_axis_name="s"),
        scratch_types=[
            pltpu.VMEM((rpw,), jnp.int32),
            pltpu.VMEM((6, 5, WIN), jnp.float32),
            pltpu.SemaphoreType.DMA,
            pltpu.SemaphoreType.DMA,
        ],
        compiler_params=pltpu.CompilerParams(use_tc_tiling_on_sc=False),
    )
    def pack(sig_hbm, dt_hbm, rad_hbm, offs_hbm, out, offs_v, buf,
             gsem, ssem):
        c = lax.axis_index("c")
        s = lax.axis_index("s")
        wid = s * info.num_cores + c
        base = wid * rpw
        pltpu.sync_copy(offs_hbm.at[pl.ds(base, rpw)], offs_v)

        def outer(g, carry):
            offv = offs_v[pl.ds(g * 16, 16)]
            gathers = {}
            scatters = {}

            def fire(j):
                b = j % 6
                if b in scatters:
                    scatters.pop(b).wait()
                off = offv[j]
                ws = pl.multiple_of(
                    jnp.minimum(jnp.bitwise_and(off, -8), wclamp), 8)
                gathers[b] = [
                    pltpu.async_copy(sig_hbm.at[pl.ds(ws, WIN)],
                                     buf.at[b, 0], gsem),
                    pltpu.async_copy(dt_hbm.at[pl.ds(ws, WIN)],
                                     buf.at[b, 1], gsem),
                    pltpu.async_copy(rad_hbm.at[:, pl.ds(ws, WIN)],
                                     buf.at[b, pl.ds(2, 3)], gsem),
                ]

            for jj in range(5):
                fire(jj)
            for j in range(16):
                if j + 5 < 16:
                    fire(j + 5)
                b = j % 6
                for h in gathers.pop(b):
                    h.wait()
                ray = base + g * 16 + j
                scatters[b] = pltpu.async_copy(buf.at[b], out.at[:, ray, :],
                                               ssem)
            for b in list(scatters):
                scatters.pop(b).wait()
            return carry

        lax.fori_loop(0, rpw // 16, outer, 0)

    return pack(sigma, dtv, rad, offs)


def kernel(sigma, radiance, dt, numsteps_in, bkg_color, inference_only):
    del inference_only
    n_rays = numsteps_in.shape[0]
    total = sigma.shape[0]
    nch = 4
    cr = n_rays // nch
    rad_t = radiance.T
    rgbs, masks = [], []
    for k in range(nch):
        ns_k = lax.slice_in_dim(numsteps_in, k * cr, (k + 1) * cr, axis=0)
        pk = _sc_pack(sigma, dt, rad_t, ns_k[:, 1], cr, total)
        rgb_k, mask_k = _render(pk, ns_k, bkg_color, total)
        rgbs.append(rgb_k)
        masks.append(mask_k)
    rgb = jnp.concatenate(rgbs, axis=0)
    mask = jnp.concatenate(masks, axis=0)
    return rgb, mask.reshape(n_rays)


# speedup vs baseline: 1.0074x; 1.0016x over previous
import functools

import numpy as np
import jax
import jax.numpy as jnp
from jax import lax
from jax.experimental import pallas as pl
from jax.experimental.pallas import tpu as pltpu
from jax.experimental.pallas import tpu_sc as plsc

WIN = 640
RB = 128
CHUNK = 128
_UPPER = np.triu(np.ones((CHUNK, CHUNK), dtype=np.float32))


def _make_render_body(total: int):
    def body(ns_ref, bkg_ref, u_ref, pk_ref, rgb_ref, mask_ref):
        ns = ns_ref[...]
        steps = ns[:, 0:1]
        off = ns[:, 1:2]
        ws = jnp.minimum(jnp.bitwise_and(off, -8),
                         jnp.bitwise_and(total - WIN, -8))
        ls = off - ws
        sig = pk_ref[0]
        dtv = pk_ref[1]
        iota = lax.broadcasted_iota(jnp.int32, (RB, WIN), 1)
        valid = (iota >= ls) & (iota < ls + steps)
        x = jnp.where(valid, jnp.maximum(sig, 0.0) * dtv, 0.0)
        u = u_ref[...]
        parts = []
        carry = jnp.zeros((RB, 1), jnp.float32)
        for k in range(WIN // CHUNK):
            ck = x[:, k * CHUNK:(k + 1) * CHUNK]
            sk = jnp.dot(ck, u, preferred_element_type=jnp.float32,
                         precision=lax.Precision.HIGHEST) + carry
            carry = sk[:, CHUNK - 1:CHUNK]
            parts.append(sk)
        s = jnp.concatenate(parts, axis=1)
        w = jnp.exp(x - s) - jnp.exp(-s)
        s_end = s[:, WIN - 1:WIN]
        xl = jnp.sum(jnp.where(iota == ls + 511, x, 0.0), axis=1, keepdims=True)
        t_bkg = jnp.exp(xl - s_end)
        mask_ref[...] = 1.0 - jnp.exp(-s_end)
        for c in range(3):
            acc = jnp.sum(w * pk_ref[2 + c], axis=1, keepdims=True)
            rgb_ref[:, c:c + 1] = acc + t_bkg * bkg_ref[0, c]

    return body


def _render(pk, ns, bkg, total):
    n_rays = ns.shape[0]
    grid = (n_rays // RB,)
    return pl.pallas_call(
        _make_render_body(total),
        grid=grid,
        in_specs=[
            pl.BlockSpec((RB, 2), lambda i: (i, 0)),
            pl.BlockSpec((1, 3), lambda i: (0, 0)),
            pl.BlockSpec((CHUNK, CHUNK), lambda i: (0, 0)),
            pl.BlockSpec((5, RB, WIN), lambda i: (0, i, 0)),
        ],
        out_specs=[
            pl.BlockSpec((RB, 3), lambda i: (i, 0)),
            pl.BlockSpec((RB, 1), lambda i: (i, 0)),
        ],
        out_shape=[
            jax.ShapeDtypeStruct((n_rays, 3), jnp.float32),
            jax.ShapeDtypeStruct((n_rays, 1), jnp.float32),
        ],
    )(ns, bkg, jnp.asarray(_UPPER), pk)


def _sc_pack(sigma, dtv, rad, offs, n_rays, total):
    info = plsc.get_sparse_core_info()
    nw = info.num_cores * info.num_subcores
    rpw = n_rays // nw
    wclamp = (total - WIN) & -8

    @functools.partial(
        pl.kernel,
        out_type=jax.ShapeDtypeStruct((5, n_rays, WIN), jnp.float32),
        mesh=plsc.VectorSubcoreMesh(core_axis_name="c", subcore_axis_name="s"),
        scratch_types=[
            pltpu.VMEM((rpw,), jnp.int32),
            pltpu.VMEM((4, 5, WIN), jnp.float32),
            pltpu.SemaphoreType.DMA,
            pltpu.SemaphoreType.DMA,
        ],
        compiler_params=pltpu.CompilerParams(use_tc_tiling_on_sc=False),
    )
    def pack(sig_hbm, dt_hbm, rad_hbm, offs_hbm, out, offs_v, buf,
             gsem, ssem):
        c = lax.axis_index("c")
        s = lax.axis_index("s")
        wid = s * info.num_cores + c
        base = wid * rpw
        pltpu.sync_copy(offs_hbm.at[pl.ds(base, rpw)], offs_v)

        def outer(g, carry):
            offv = offs_v[pl.ds(g * 16, 16)]
            gathers = {}
            scatters = {}

            def fire(j):
                b = j % 4
                if b in scatters:
                    scatters.pop(b).wait()
                off = offv[j]
                ws = pl.multiple_of(
                    jnp.minimum(jnp.bitwise_and(off, -8), wclamp), 8)
                gathers[b] = [
                    pltpu.async_copy(sig_hbm.at[pl.ds(ws, WIN)],
                                     buf.at[b, 0], gsem),
                    pltpu.async_copy(dt_hbm.at[pl.ds(ws, WIN)],
                                     buf.at[b, 1], gsem),
                    pltpu.async_copy(rad_hbm.at[:, pl.ds(ws, WIN)],
                                     buf.at[b, pl.ds(2, 3)], gsem),
                ]

            fire(0)
            fire(1)
            fire(2)
            for j in range(16):
                if j + 3 < 16:
                    fire(j + 3)
                b = j % 4
                for h in gathers.pop(b):
                    h.wait()
                ray = base + g * 16 + j
                scatters[b] = pltpu.async_copy(buf.at[b], out.at[:, ray, :],
                                               ssem)
            for b in list(scatters):
                scatters.pop(b).wait()
            return carry

        lax.fori_loop(0, rpw // 16, outer, 0)

    return pack(sigma, dtv, rad, offs)


def kernel(sigma, radiance, dt, numsteps_in, bkg_color, inference_only):
    del inference_only
    n_rays = numsteps_in.shape[0]
    total = sigma.shape[0]
    nch = 4
    cr = n_rays // nch
    rad_t = radiance.T
    rgbs, masks = [], []
    for k in range(nch):
        ns_k = lax.slice_in_dim(numsteps_in, k * cr, (k + 1) * cr, axis=0)
        pk = _sc_pack(sigma, dt, rad_t, ns_k[:, 1], cr, total)
        rgb_k, mask_k = _render(pk, ns_k, bkg_color, total)
        rgbs.append(rgb_k)
        masks.append(mask_k)
    rgb = jnp.concatenate(rgbs, axis=0)
    mask = jnp.concatenate(masks, axis=0)
    return rgb, mask.reshape(n_rays)
